# Initial kernel scaffold; baseline (speedup 1.0000x reference)
import functools

import jax
import jax.numpy as jnp
from jax import lax
from jax.experimental import pallas as pl
from jax.experimental.pallas import tpu as pltpu
from jax.experimental.pallas import tpu_sc as plsc

N = 10000
E = 320000
D_IN = 128
D_HID = 64

NC = 2
NS = 16
NW = NC * NS
NPAD = 10240
ROWS_PER_TILE = NPAD // NS
CHUNK = 128
CPT = 80
EPT = CPT * CHUNK
EPAD = NW * EPT
KBUF = 5
BR = 256

_mesh = plsc.VectorSubcoreMesh(
    core_axis_name="c", subcore_axis_name="s", num_cores=NC, num_subcores=NS)


@functools.partial(
    pl.kernel,
    out_type=jax.ShapeDtypeStruct((NC, NPAD), jnp.float32),
    mesh=_mesh,
    scratch_types=[
        pltpu.VMEM((CPT, CHUNK), jnp.int32),
        pltpu.VMEM((CHUNK,), jnp.float32),
        pltpu.VMEM((ROWS_PER_TILE,), jnp.float32),
        pltpu.VMEM_SHARED((NPAD,), jnp.float32),
    ],
)
def _deg_kernel(dstp_hbm, out_hbm, dst_v, ones_v, zfill_v, acc_sh):
    cid = lax.axis_index("c")
    sid = lax.axis_index("s")
    wid = cid * NS + sid
    pltpu.sync_copy(dstp_hbm.at[wid], dst_v)

    def fill_ones(i, _):
        ones_v[pl.ds(i * 16, 16)] = jnp.ones((16,), jnp.float32)
        return 0
    lax.fori_loop(0, CHUNK // 16, fill_ones, 0)

    def fill_zero(i, _):
        zfill_v[pl.ds(i * 16, 16)] = jnp.zeros((16,), jnp.float32)
        return 0
    lax.fori_loop(0, ROWS_PER_TILE // 16, fill_zero, 0)
    row0 = sid * ROWS_PER_TILE
    pltpu.sync_copy(zfill_v, acc_sh.at[pl.ds(row0, ROWS_PER_TILE)])
    plsc.subcore_barrier()

    def scatter_ones(ch, _):
        pltpu.sync_copy(ones_v, acc_sh.at[dst_v.at[ch]], add=True)
        return 0
    lax.fori_loop(0, CPT, scatter_ones, 0)
    plsc.subcore_barrier()
    pltpu.sync_copy(acc_sh.at[pl.ds(row0, ROWS_PER_TILE)],
                    out_hbm.at[cid, pl.ds(row0, ROWS_PER_TILE)])


@functools.partial(
    pl.kernel,
    out_type=jax.ShapeDtypeStruct((NC, NPAD, D_HID), jnp.float32),
    mesh=_mesh,
    scratch_types=[
        pltpu.VMEM((CPT, CHUNK), jnp.int32),
        pltpu.VMEM((CPT, CHUNK), jnp.int32),
        pltpu.VMEM((KBUF, CHUNK, D_HID), jnp.float32),
        pltpu.VMEM_SHARED((NPAD, D_HID), jnp.float32),
        pltpu.SemaphoreType.DMA,
        pltpu.SemaphoreType.DMA,
    ],
)
def _prop_kernel(z_hbm, srcp_hbm, dstp_hbm, out_hbm,
                 src_v, dst_v, bufs, acc_sh, sg, ss):
    cid = lax.axis_index("c")
    sid = lax.axis_index("s")
    wid = cid * NS + sid
    pltpu.sync_copy(srcp_hbm.at[wid], src_v)
    pltpu.sync_copy(dstp_hbm.at[wid], dst_v)
    row0 = sid * ROWS_PER_TILE
    pltpu.sync_copy(z_hbm.at[pl.ds(row0, ROWS_PER_TILE)],
                    acc_sh.at[pl.ds(row0, ROWS_PER_TILE)])
    plsc.subcore_barrier()

    def round_body(r, _):
        base = r * KBUF
        gh = [pltpu.async_copy(z_hbm.at[src_v.at[base + b]], bufs.at[b], sg)
              for b in range(KBUF)]
        for h in gh:
            h.wait()
        sh = [pltpu.async_copy(bufs.at[b], acc_sh.at[dst_v.at[base + b]],
                               ss, add=True)
              for b in range(KBUF)]
        for h in sh:
            h.wait()
        return 0
    lax.fori_loop(0, CPT // KBUF, round_body, 0)
    plsc.subcore_barrier()
    pltpu.sync_copy(acc_sh.at[pl.ds(row0, ROWS_PER_TILE)],
                    out_hbm.at[cid, pl.ds(row0, ROWS_PER_TILE)])


def _mm1_body(x_ref, w_ref, d0_ref, d1_ref, z_ref, dis_ref):
    dis = lax.rsqrt(d0_ref[...] + d1_ref[...] + 1.0)
    z_ref[...] = jnp.dot(x_ref[...], w_ref[...],
                         preferred_element_type=jnp.float32) * dis
    dis_ref[...] = dis


def _combine_body(p0_ref, p1_ref, z0_ref, dis_ref, b1_ref, z1_ref):
    dis = dis_ref[...]
    t = (p0_ref[...] + p1_ref[...] - z0_ref[...]) * dis + b1_ref[...]
    z1_ref[...] = jnp.maximum(t, 0.0) * dis


def _final_body(q0_ref, q1_ref, z1_ref, dis_ref, wm_ref, bm_ref,
                wl_ref, bl_ref, mu_ref, ls_ref):
    g = (q0_ref[...] + q1_ref[...] - z1_ref[...]) * dis_ref[...]
    mu_ref[...] = jnp.dot(g, wm_ref[...],
                          preferred_element_type=jnp.float32) + bm_ref[...]
    ls_ref[...] = jnp.dot(g, wl_ref[...],
                          preferred_element_type=jnp.float32) + bl_ref[...]


def _row_spec(d):
    return pl.BlockSpec((BR, d), lambda i: (i, 0))


def _full_spec(r, c):
    return pl.BlockSpec((r, c), lambda i: (0, 0))


@jax.jit
def kernel(x, edge_index, W1, b1, W_mu, b_mu, W_ls, b_ls):
    src = edge_index[0].astype(jnp.int32)
    dst = edge_index[1].astype(jnp.int32)
    padlen = EPAD - E
    srcp = jnp.concatenate(
        [src, jnp.full((padlen,), N, jnp.int32)]).reshape(NW, CPT, CHUNK)
    dstp = jnp.concatenate(
        [dst, jnp.full((padlen,), N, jnp.int32)]).reshape(NW, CPT, CHUNK)
    xpad = jnp.pad(x, ((0, NPAD - N), (0, 0)))

    degp = _deg_kernel(dstp)
    d0 = degp[0][:, None]
    d1 = degp[1][:, None]

    grid = (NPAD // BR,)
    z0, dis = pl.pallas_call(
        _mm1_body,
        grid=grid,
        in_specs=[_row_spec(D_IN), _full_spec(D_IN, D_HID),
                  _row_spec(1), _row_spec(1)],
        out_specs=[_row_spec(D_HID), _row_spec(1)],
        out_shape=[jax.ShapeDtypeStruct((NPAD, D_HID), jnp.float32),
                   jax.ShapeDtypeStruct((NPAD, 1), jnp.float32)],
    )(xpad, W1, d0, d1)

    p = _prop_kernel(z0, srcp, dstp)

    z1 = pl.pallas_call(
        _combine_body,
        grid=grid,
        in_specs=[_row_spec(D_HID), _row_spec(D_HID), _row_spec(D_HID),
                  _row_spec(1), _full_spec(1, D_HID)],
        out_specs=_row_spec(D_HID),
        out_shape=jax.ShapeDtypeStruct((NPAD, D_HID), jnp.float32),
    )(p[0], p[1], z0, dis, b1[None, :])

    q = _prop_kernel(z1, srcp, dstp)

    mu, ls = pl.pallas_call(
        _final_body,
        grid=grid,
        in_specs=[_row_spec(D_HID), _row_spec(D_HID), _row_spec(D_HID),
                  _row_spec(1), _full_spec(D_HID, D_HID),
                  _full_spec(1, D_HID), _full_spec(D_HID, D_HID),
                  _full_spec(1, D_HID)],
        out_specs=[_row_spec(D_HID), _row_spec(D_HID)],
        out_shape=[jax.ShapeDtypeStruct((NPAD, D_HID), jnp.float32),
                   jax.ShapeDtypeStruct((NPAD, D_HID), jnp.float32)],
    )(q[0], q[1], z1, dis, W_mu, b_mu[None, :], W_ls, b_ls[None, :])

    return mu[:N], ls[:N]


# speedup vs baseline: 17.3909x; 17.3909x over previous
import functools

import jax
import jax.numpy as jnp
from jax import lax
from jax.experimental import pallas as pl
from jax.experimental.pallas import tpu as pltpu
from jax.experimental.pallas import tpu_sc as plsc

N = 10000
E = 320000
D_IN = 128
D_HID = 64

NC = 2
NS = 16
NW = NC * NS
NPAD = 10240
ROWS_PER_TILE = NPAD // NS
CHUNK = 128
CPT = 80
EPT = CPT * CHUNK
EPAD = NW * EPT
KBUF = 5
BR = 256

_mesh = plsc.VectorSubcoreMesh(
    core_axis_name="c", subcore_axis_name="s", num_cores=NC, num_subcores=NS)
_sc_params = pltpu.CompilerParams(use_tc_tiling_on_sc=False)


@functools.partial(
    pl.kernel,
    out_type=jax.ShapeDtypeStruct((NC, NPAD), jnp.float32),
    mesh=_mesh,
    scratch_types=[
        pltpu.VMEM((CPT, CHUNK), jnp.int32),
        pltpu.VMEM((CHUNK,), jnp.float32),
        pltpu.VMEM((ROWS_PER_TILE,), jnp.float32),
        pltpu.VMEM_SHARED((NPAD,), jnp.float32),
    ],
    compiler_params=_sc_params,
)
def _deg_kernel(dstp_hbm, out_hbm, dst_v, ones_v, zfill_v, acc_sh):
    cid = lax.axis_index("c")
    sid = lax.axis_index("s")
    wid = cid * NS + sid
    pltpu.sync_copy(dstp_hbm.at[wid], dst_v)

    def fill_ones(i, _):
        ones_v[pl.ds(i * 16, 16)] = jnp.ones((16,), jnp.float32)
        return 0
    lax.fori_loop(0, CHUNK // 16, fill_ones, 0)

    def fill_zero(i, _):
        zfill_v[pl.ds(i * 16, 16)] = jnp.zeros((16,), jnp.float32)
        return 0
    lax.fori_loop(0, ROWS_PER_TILE // 16, fill_zero, 0)
    row0 = sid * ROWS_PER_TILE
    pltpu.sync_copy(zfill_v, acc_sh.at[pl.ds(row0, ROWS_PER_TILE)])
    plsc.subcore_barrier()

    def scatter_ones(ch, _):
        pltpu.sync_copy(ones_v, acc_sh.at[dst_v.at[ch]], add=True)
        return 0
    lax.fori_loop(0, CPT, scatter_ones, 0)
    plsc.subcore_barrier()
    pltpu.sync_copy(acc_sh.at[pl.ds(row0, ROWS_PER_TILE)],
                    out_hbm.at[cid, pl.ds(row0, ROWS_PER_TILE)])


@functools.partial(
    pl.kernel,
    out_type=jax.ShapeDtypeStruct((NC, NPAD, D_HID), jnp.float32),
    mesh=_mesh,
    scratch_types=[
        pltpu.VMEM((CPT, CHUNK), jnp.int32),
        pltpu.VMEM((CPT, CHUNK), jnp.int32),
        pltpu.VMEM((KBUF, CHUNK, D_HID), jnp.float32),
        pltpu.VMEM_SHARED((NPAD, D_HID), jnp.float32),
        pltpu.SemaphoreType.DMA,
        pltpu.SemaphoreType.DMA,
    ],
    compiler_params=_sc_params,
)
def _prop_kernel(z_hbm, srcp_hbm, dstp_hbm, out_hbm,
                 src_v, dst_v, bufs, acc_sh, sg, ss):
    cid = lax.axis_index("c")
    sid = lax.axis_index("s")
    wid = cid * NS + sid
    pltpu.sync_copy(srcp_hbm.at[wid], src_v)
    pltpu.sync_copy(dstp_hbm.at[wid], dst_v)
    row0 = sid * ROWS_PER_TILE
    pltpu.sync_copy(z_hbm.at[pl.ds(row0, ROWS_PER_TILE)],
                    acc_sh.at[pl.ds(row0, ROWS_PER_TILE)])
    plsc.subcore_barrier()

    def round_body(r, _):
        base = r * KBUF
        gh = [pltpu.async_copy(z_hbm.at[src_v.at[base + b]], bufs.at[b], sg)
              for b in range(KBUF)]
        for h in gh:
            h.wait()
        sh = [pltpu.async_copy(bufs.at[b], acc_sh.at[dst_v.at[base + b]],
                               ss, add=True)
              for b in range(KBUF)]
        for h in sh:
            h.wait()
        return 0
    lax.fori_loop(0, CPT // KBUF, round_body, 0)
    plsc.subcore_barrier()
    pltpu.sync_copy(acc_sh.at[pl.ds(row0, ROWS_PER_TILE)],
                    out_hbm.at[cid, pl.ds(row0, ROWS_PER_TILE)])


def _mm1_body(x_ref, w_ref, d0_ref, d1_ref, z_ref, dis_ref):
    dis = lax.rsqrt(d0_ref[...] + d1_ref[...] + 1.0)
    z_ref[...] = jnp.dot(x_ref[...], w_ref[...],
                         preferred_element_type=jnp.float32) * dis
    dis_ref[...] = dis


def _combine_body(p0_ref, p1_ref, z0_ref, dis_ref, b1_ref, z1_ref):
    dis = dis_ref[...]
    t = (p0_ref[...] + p1_ref[...] - z0_ref[...]) * dis + b1_ref[...]
    z1_ref[...] = jnp.maximum(t, 0.0) * dis


def _final_body(q0_ref, q1_ref, z1_ref, dis_ref, wm_ref, bm_ref,
                wl_ref, bl_ref, mu_ref, ls_ref):
    g = (q0_ref[...] + q1_ref[...] - z1_ref[...]) * dis_ref[...]
    mu_ref[...] = jnp.dot(g, wm_ref[...],
                          preferred_element_type=jnp.float32) + bm_ref[...]
    ls_ref[...] = jnp.dot(g, wl_ref[...],
                          preferred_element_type=jnp.float32) + bl_ref[...]


def _row_spec(d):
    return pl.BlockSpec((BR, d), lambda i: (i, 0))


def _full_spec(r, c):
    return pl.BlockSpec((r, c), lambda i: (0, 0))


@jax.jit
def kernel(x, edge_index, W1, b1, W_mu, b_mu, W_ls, b_ls):
    src = edge_index[0].astype(jnp.int32)
    dst = edge_index[1].astype(jnp.int32)
    padlen = EPAD - E
    srcp = jnp.concatenate(
        [src, jnp.full((padlen,), N, jnp.int32)]).reshape(NW, CPT, CHUNK)
    dstp = jnp.concatenate(
        [dst, jnp.full((padlen,), N, jnp.int32)]).reshape(NW, CPT, CHUNK)
    xpad = jnp.pad(x, ((0, NPAD - N), (0, 0)))

    degp = _deg_kernel(dstp)
    d0 = degp[0][:, None]
    d1 = degp[1][:, None]

    grid = (NPAD // BR,)
    z0, dis = pl.pallas_call(
        _mm1_body,
        grid=grid,
        in_specs=[_row_spec(D_IN), _full_spec(D_IN, D_HID),
                  _row_spec(1), _row_spec(1)],
        out_specs=[_row_spec(D_HID), _row_spec(1)],
        out_shape=[jax.ShapeDtypeStruct((NPAD, D_HID), jnp.float32),
                   jax.ShapeDtypeStruct((NPAD, 1), jnp.float32)],
    )(xpad, W1, d0, d1)

    p = _prop_kernel(z0, srcp, dstp)

    z1 = pl.pallas_call(
        _combine_body,
        grid=grid,
        in_specs=[_row_spec(D_HID), _row_spec(D_HID), _row_spec(D_HID),
                  _row_spec(1), _full_spec(1, D_HID)],
        out_specs=_row_spec(D_HID),
        out_shape=jax.ShapeDtypeStruct((NPAD, D_HID), jnp.float32),
    )(p[0], p[1], z0, dis, b1[None, :])

    q = _prop_kernel(z1, srcp, dstp)

    mu, ls = pl.pallas_call(
        _final_body,
        grid=grid,
        in_specs=[_row_spec(D_HID), _row_spec(D_HID), _row_spec(D_HID),
                  _row_spec(1), _full_spec(D_HID, D_HID),
                  _full_spec(1, D_HID), _full_spec(D_HID, D_HID),
                  _full_spec(1, D_HID)],
        out_specs=[_row_spec(D_HID), _row_spec(D_HID)],
        out_shape=[jax.ShapeDtypeStruct((NPAD, D_HID), jnp.float32),
                   jax.ShapeDtypeStruct((NPAD, D_HID), jnp.float32)],
    )(q[0], q[1], z1, dis, W_mu, b_mu[None, :], W_ls, b_ls[None, :])

    return mu[:N], ls[:N]


# speedup vs baseline: 28.9426x; 1.6642x over previous
import functools

import jax
import jax.numpy as jnp
from jax import lax
from jax.experimental import pallas as pl
from jax.experimental.pallas import tpu as pltpu
from jax.experimental.pallas import tpu_sc as plsc

N = 10000
E = 320000
D_IN = 128
D_HID = 64

NC = 2
NS = 16
NW = NC * NS
NPAD = 10240
ROWS_PER_TILE = NPAD // NS
CHUNK = 128
CPT = 80
EPT = CPT * CHUNK
EPAD = NW * EPT
KBUF = 2
BR = 256

_mesh = plsc.VectorSubcoreMesh(
    core_axis_name="c", subcore_axis_name="s", num_cores=NC, num_subcores=NS)
_sc_params = pltpu.CompilerParams(use_tc_tiling_on_sc=False)


@functools.partial(
    pl.kernel,
    out_type=jax.ShapeDtypeStruct((NC, NPAD), jnp.float32),
    mesh=_mesh,
    scratch_types=[
        pltpu.VMEM((CPT, CHUNK), jnp.int32),
        pltpu.VMEM((CHUNK,), jnp.float32),
        pltpu.VMEM((ROWS_PER_TILE,), jnp.float32),
        pltpu.VMEM_SHARED((NPAD,), jnp.float32),
    ],
    compiler_params=_sc_params,
)
def _deg_kernel(dstp_hbm, out_hbm, dst_v, ones_v, zfill_v, acc_sh):
    cid = lax.axis_index("c")
    sid = lax.axis_index("s")
    wid = cid * NS + sid
    pltpu.sync_copy(dstp_hbm.at[wid], dst_v)

    def fill_ones(i, _):
        ones_v[pl.ds(i * 16, 16)] = jnp.ones((16,), jnp.float32)
        return 0
    lax.fori_loop(0, CHUNK // 16, fill_ones, 0)

    def fill_zero(i, _):
        zfill_v[pl.ds(i * 16, 16)] = jnp.zeros((16,), jnp.float32)
        return 0
    lax.fori_loop(0, ROWS_PER_TILE // 16, fill_zero, 0)
    row0 = sid * ROWS_PER_TILE
    pltpu.sync_copy(zfill_v, acc_sh.at[pl.ds(row0, ROWS_PER_TILE)])
    plsc.subcore_barrier()

    def scatter_ones(ch, _):
        pltpu.sync_copy(ones_v, acc_sh.at[dst_v.at[ch]], add=True)
        return 0
    lax.fori_loop(0, CPT, scatter_ones, 0)
    plsc.subcore_barrier()
    pltpu.sync_copy(acc_sh.at[pl.ds(row0, ROWS_PER_TILE)],
                    out_hbm.at[cid, pl.ds(row0, ROWS_PER_TILE)])


@functools.partial(
    pl.kernel,
    out_type=jax.ShapeDtypeStruct((NC, NPAD, D_HID), jnp.float32),
    mesh=_mesh,
    scratch_types=[
        pltpu.VMEM((CPT, CHUNK), jnp.int32),
        pltpu.VMEM((CPT, CHUNK), jnp.int32),
        pltpu.VMEM((KBUF, CHUNK, D_HID), jnp.float32),
        pltpu.VMEM_SHARED((NPAD, D_HID), jnp.float32),
        pltpu.VMEM_SHARED((NPAD, D_HID), jnp.float32),
        pltpu.SemaphoreType.DMA,
        pltpu.SemaphoreType.DMA,
    ],
    compiler_params=_sc_params,
)
def _prop_kernel(z_hbm, srcp_hbm, dstp_hbm, out_hbm,
                 src_v, dst_v, bufs, acc_sh, z_sh, sg, ss):
    cid = lax.axis_index("c")
    sid = lax.axis_index("s")
    wid = cid * NS + sid
    pltpu.sync_copy(srcp_hbm.at[wid], src_v)
    pltpu.sync_copy(dstp_hbm.at[wid], dst_v)
    row0 = sid * ROWS_PER_TILE
    pltpu.sync_copy(z_hbm.at[pl.ds(row0, ROWS_PER_TILE)],
                    z_sh.at[pl.ds(row0, ROWS_PER_TILE)])
    pltpu.sync_copy(z_hbm.at[pl.ds(row0, ROWS_PER_TILE)],
                    acc_sh.at[pl.ds(row0, ROWS_PER_TILE)])
    plsc.subcore_barrier()

    def round_body(r, _):
        base = r * KBUF
        gh = [pltpu.async_copy(z_sh.at[src_v.at[base + b]], bufs.at[b], sg)
              for b in range(KBUF)]
        for h in gh:
            h.wait()
        sh = [pltpu.async_copy(bufs.at[b], acc_sh.at[dst_v.at[base + b]],
                               ss, add=True)
              for b in range(KBUF)]
        for h in sh:
            h.wait()
        return 0
    lax.fori_loop(0, CPT // KBUF, round_body, 0)
    plsc.subcore_barrier()
    for piece in range(ROWS_PER_TILE // CHUNK):
        r0 = row0 + piece * CHUNK
        b = piece % KBUF
        pltpu.sync_copy(acc_sh.at[pl.ds(r0, CHUNK)], bufs.at[b])
        pltpu.sync_copy(bufs.at[b], out_hbm.at[cid, pl.ds(r0, CHUNK)])


def _mm1_body(x_ref, w_ref, d0_ref, d1_ref, z_ref, dis_ref):
    dis = lax.rsqrt(d0_ref[...] + d1_ref[...] + 1.0)
    z_ref[...] = jnp.dot(x_ref[...], w_ref[...],
                         preferred_element_type=jnp.float32) * dis
    dis_ref[...] = dis


def _combine_body(p0_ref, p1_ref, z0_ref, dis_ref, b1_ref, z1_ref):
    dis = dis_ref[...]
    t = (p0_ref[...] + p1_ref[...] - z0_ref[...]) * dis + b1_ref[...]
    z1_ref[...] = jnp.maximum(t, 0.0) * dis


def _final_body(q0_ref, q1_ref, z1_ref, dis_ref, wm_ref, bm_ref,
                wl_ref, bl_ref, mu_ref, ls_ref):
    g = (q0_ref[...] + q1_ref[...] - z1_ref[...]) * dis_ref[...]
    mu_ref[...] = jnp.dot(g, wm_ref[...],
                          preferred_element_type=jnp.float32) + bm_ref[...]
    ls_ref[...] = jnp.dot(g, wl_ref[...],
                          preferred_element_type=jnp.float32) + bl_ref[...]


def _row_spec(d):
    return pl.BlockSpec((BR, d), lambda i: (i, 0))


def _full_spec(r, c):
    return pl.BlockSpec((r, c), lambda i: (0, 0))


@jax.jit
def kernel(x, edge_index, W1, b1, W_mu, b_mu, W_ls, b_ls):
    src = edge_index[0].astype(jnp.int32)
    dst = edge_index[1].astype(jnp.int32)
    padlen = EPAD - E
    srcp = jnp.concatenate(
        [src, jnp.full((padlen,), N, jnp.int32)]).reshape(NW, CPT, CHUNK)
    dstp = jnp.concatenate(
        [dst, jnp.full((padlen,), N, jnp.int32)]).reshape(NW, CPT, CHUNK)
    xpad = jnp.pad(x, ((0, NPAD - N), (0, 0)))

    degp = _deg_kernel(dstp)
    d0 = degp[0][:, None]
    d1 = degp[1][:, None]

    grid = (NPAD // BR,)
    z0, dis = pl.pallas_call(
        _mm1_body,
        grid=grid,
        in_specs=[_row_spec(D_IN), _full_spec(D_IN, D_HID),
                  _row_spec(1), _row_spec(1)],
        out_specs=[_row_spec(D_HID), _row_spec(1)],
        out_shape=[jax.ShapeDtypeStruct((NPAD, D_HID), jnp.float32),
                   jax.ShapeDtypeStruct((NPAD, 1), jnp.float32)],
    )(xpad, W1, d0, d1)

    p = _prop_kernel(z0, srcp, dstp)

    z1 = pl.pallas_call(
        _combine_body,
        grid=grid,
        in_specs=[_row_spec(D_HID), _row_spec(D_HID), _row_spec(D_HID),
                  _row_spec(1), _full_spec(1, D_HID)],
        out_specs=_row_spec(D_HID),
        out_shape=jax.ShapeDtypeStruct((NPAD, D_HID), jnp.float32),
    )(p[0], p[1], z0, dis, b1[None, :])

    q = _prop_kernel(z1, srcp, dstp)

    mu, ls = pl.pallas_call(
        _final_body,
        grid=grid,
        in_specs=[_row_spec(D_HID), _row_spec(D_HID), _row_spec(D_HID),
                  _row_spec(1), _full_spec(D_HID, D_HID),
                  _full_spec(1, D_HID), _full_spec(D_HID, D_HID),
                  _full_spec(1, D_HID)],
        out_specs=[_row_spec(D_HID), _row_spec(D_HID)],
        out_shape=[jax.ShapeDtypeStruct((NPAD, D_HID), jnp.float32),
                   jax.ShapeDtypeStruct((NPAD, D_HID), jnp.float32)],
    )(q[0], q[1], z1, dis, W_mu, b_mu[None, :], W_ls, b_ls[None, :])

    return mu[:N], ls[:N]


# speedup vs baseline: 40.2070x; 1.3892x over previous
import functools

import jax
import jax.numpy as jnp
from jax import lax
from jax.experimental import pallas as pl
from jax.experimental.pallas import tpu as pltpu
from jax.experimental.pallas import tpu_sc as plsc

N = 10000
E = 320000
D_IN = 128
D_HID = 64

NC = 2
NS = 16
NW = NC * NS
NPAD = 10240
ROWS_PER_TILE = NPAD // NS
CHUNK = 128
CPT = 81
EPT = CPT * CHUNK
EPAD = NW * EPT
KBUF = 3
BR = 1024

_mesh = plsc.VectorSubcoreMesh(
    core_axis_name="c", subcore_axis_name="s", num_cores=NC, num_subcores=NS)
_sc_params = pltpu.CompilerParams(use_tc_tiling_on_sc=False)


@functools.partial(
    pl.kernel,
    out_type=jax.ShapeDtypeStruct((NC, NPAD), jnp.float32),
    mesh=_mesh,
    scratch_types=[
        pltpu.VMEM((CPT, CHUNK), jnp.int32),
        pltpu.VMEM((CHUNK,), jnp.float32),
        pltpu.VMEM((ROWS_PER_TILE,), jnp.float32),
        pltpu.VMEM_SHARED((NPAD,), jnp.float32),
    ],
    compiler_params=_sc_params,
)
def _deg_kernel(dstp_hbm, out_hbm, dst_v, ones_v, zfill_v, acc_sh):
    cid = lax.axis_index("c")
    sid = lax.axis_index("s")
    wid = cid * NS + sid
    pltpu.sync_copy(dstp_hbm.at[wid], dst_v)

    def fill_ones(i, _):
        ones_v[pl.ds(i * 16, 16)] = jnp.ones((16,), jnp.float32)
        return 0
    lax.fori_loop(0, CHUNK // 16, fill_ones, 0)

    def fill_zero(i, _):
        zfill_v[pl.ds(i * 16, 16)] = jnp.zeros((16,), jnp.float32)
        return 0
    lax.fori_loop(0, ROWS_PER_TILE // 16, fill_zero, 0)
    row0 = sid * ROWS_PER_TILE
    pltpu.sync_copy(zfill_v, acc_sh.at[pl.ds(row0, ROWS_PER_TILE)])
    plsc.subcore_barrier()

    def scatter_ones(ch, _):
        pltpu.sync_copy(ones_v, acc_sh.at[dst_v.at[ch]], add=True)
        return 0
    lax.fori_loop(0, CPT, scatter_ones, 0)
    plsc.subcore_barrier()
    pltpu.sync_copy(acc_sh.at[pl.ds(row0, ROWS_PER_TILE)],
                    out_hbm.at[cid, pl.ds(row0, ROWS_PER_TILE)])


@functools.partial(
    pl.kernel,
    out_type=jax.ShapeDtypeStruct((NC, NPAD, D_HID), jnp.float32),
    mesh=_mesh,
    scratch_types=[
        pltpu.VMEM((CPT, CHUNK), jnp.int32),
        pltpu.VMEM((CPT, CHUNK), jnp.int32),
        pltpu.VMEM((KBUF, CHUNK, D_HID), jnp.float32),
        pltpu.VMEM_SHARED((NPAD, D_HID), jnp.float32),
        pltpu.VMEM_SHARED((NPAD, D_HID), jnp.float32),
        pltpu.SemaphoreType.DMA,
        pltpu.SemaphoreType.DMA,
    ],
    compiler_params=_sc_params,
)
def _prop_kernel(z_hbm, srcp_hbm, dstp_hbm, out_hbm,
                 src_v, dst_v, bufs, acc_sh, z_sh, sg, ss):
    cid = lax.axis_index("c")
    sid = lax.axis_index("s")
    wid = cid * NS + sid
    pltpu.sync_copy(srcp_hbm.at[wid], src_v)
    pltpu.sync_copy(dstp_hbm.at[wid], dst_v)
    row0 = sid * ROWS_PER_TILE
    pltpu.sync_copy(z_hbm.at[pl.ds(row0, ROWS_PER_TILE)],
                    z_sh.at[pl.ds(row0, ROWS_PER_TILE)])
    pltpu.sync_copy(z_hbm.at[pl.ds(row0, ROWS_PER_TILE)],
                    acc_sh.at[pl.ds(row0, ROWS_PER_TILE)])
    plsc.subcore_barrier()

    for j in range(KBUF):
        pltpu.async_copy(z_sh.at[src_v.at[j]], bufs.at[j], sg)

    def round_body(p, _):
        base = p * KBUF
        for j in range(KBUF):
            ch = base + j
            pltpu.make_async_copy(z_sh.at[src_v.at[ch]],
                                  bufs.at[j], sg).wait()
            pltpu.async_copy(bufs.at[j], acc_sh.at[dst_v.at[ch]],
                             ss, add=True)
        for j in range(KBUF):
            ch = base + j
            ch2 = ch + KBUF
            pltpu.make_async_copy(bufs.at[j], acc_sh.at[dst_v.at[ch]],
                                  ss).wait()

            @pl.when(ch2 < CPT)
            def _():
                pltpu.async_copy(z_sh.at[src_v.at[ch2]], bufs.at[j], sg)
        return 0
    lax.fori_loop(0, CPT // KBUF, round_body, 0)
    plsc.subcore_barrier()
    for piece in range(ROWS_PER_TILE // CHUNK):
        r0 = row0 + piece * CHUNK
        b = piece % KBUF
        pltpu.sync_copy(acc_sh.at[pl.ds(r0, CHUNK)], bufs.at[b])
        pltpu.sync_copy(bufs.at[b], out_hbm.at[cid, pl.ds(r0, CHUNK)])


def _mm1_body(x_ref, w_ref, d0_ref, d1_ref, z_ref, dis_ref):
    dis = lax.rsqrt(d0_ref[...] + d1_ref[...] + 1.0)
    z_ref[...] = jnp.dot(x_ref[...], w_ref[...],
                         preferred_element_type=jnp.float32) * dis
    dis_ref[...] = dis


def _combine_body(p0_ref, p1_ref, z0_ref, dis_ref, b1_ref, z1_ref):
    dis = dis_ref[...]
    t = (p0_ref[...] + p1_ref[...] - z0_ref[...]) * dis + b1_ref[...]
    z1_ref[...] = jnp.maximum(t, 0.0) * dis


def _final_body(q0_ref, q1_ref, z1_ref, dis_ref, wm_ref, bm_ref,
                wl_ref, bl_ref, mu_ref, ls_ref):
    g = (q0_ref[...] + q1_ref[...] - z1_ref[...]) * dis_ref[...]
    mu_ref[...] = jnp.dot(g, wm_ref[...],
                          preferred_element_type=jnp.float32) + bm_ref[...]
    ls_ref[...] = jnp.dot(g, wl_ref[...],
                          preferred_element_type=jnp.float32) + bl_ref[...]


def _row_spec(d):
    return pl.BlockSpec((BR, d), lambda i: (i, 0))


def _full_spec(r, c):
    return pl.BlockSpec((r, c), lambda i: (0, 0))


@jax.jit
def kernel(x, edge_index, W1, b1, W_mu, b_mu, W_ls, b_ls):
    src = edge_index[0].astype(jnp.int32)
    dst = edge_index[1].astype(jnp.int32)
    padlen = EPAD - E
    srcp = jnp.concatenate(
        [src, jnp.full((padlen,), N, jnp.int32)]).reshape(NW, CPT, CHUNK)
    dstp = jnp.concatenate(
        [dst, jnp.full((padlen,), N, jnp.int32)]).reshape(NW, CPT, CHUNK)
    xpad = jnp.pad(x, ((0, NPAD - N), (0, 0)))

    degp = _deg_kernel(dstp)
    d0 = degp[0][:, None]
    d1 = degp[1][:, None]

    grid = (NPAD // BR,)
    z0, dis = pl.pallas_call(
        _mm1_body,
        grid=grid,
        in_specs=[_row_spec(D_IN), _full_spec(D_IN, D_HID),
                  _row_spec(1), _row_spec(1)],
        out_specs=[_row_spec(D_HID), _row_spec(1)],
        out_shape=[jax.ShapeDtypeStruct((NPAD, D_HID), jnp.float32),
                   jax.ShapeDtypeStruct((NPAD, 1), jnp.float32)],
    )(xpad, W1, d0, d1)

    p = _prop_kernel(z0, srcp, dstp)

    z1 = pl.pallas_call(
        _combine_body,
        grid=grid,
        in_specs=[_row_spec(D_HID), _row_spec(D_HID), _row_spec(D_HID),
                  _row_spec(1), _full_spec(1, D_HID)],
        out_specs=_row_spec(D_HID),
        out_shape=jax.ShapeDtypeStruct((NPAD, D_HID), jnp.float32),
    )(p[0], p[1], z0, dis, b1[None, :])

    q = _prop_kernel(z1, srcp, dstp)

    mu, ls = pl.pallas_call(
        _final_body,
        grid=grid,
        in_specs=[_row_spec(D_HID), _row_spec(D_HID), _row_spec(D_HID),
                  _row_spec(1), _full_spec(D_HID, D_HID),
                  _full_spec(1, D_HID), _full_spec(D_HID, D_HID),
                  _full_spec(1, D_HID)],
        out_specs=[_row_spec(D_HID), _row_spec(D_HID)],
        out_shape=[jax.ShapeDtypeStruct((NPAD, D_HID), jnp.float32),
                   jax.ShapeDtypeStruct((NPAD, D_HID), jnp.float32)],
    )(q[0], q[1], z1, dis, W_mu, b_mu[None, :], W_ls, b_ls[None, :])

    return mu[:N], ls[:N]
